# baseline (speedup 1.0000x reference)
import functools

import jax
import jax.numpy as jnp
from jax import lax
from jax.experimental import pallas as pl
from jax.experimental.pallas import tpu as pltpu
from jax.experimental.pallas import tpu_sc as plsc

L = 16


def _floor_i32(v):
    i = v.astype(jnp.int32)
    return jnp.where(i.astype(jnp.float32) > v, i - 1, i)


def _make_sc_call(N, C, H, W, NC, NS, CHUNK):
    P = N * H * W
    NW = NC * NS
    PPW = P // NW
    NCHUNKS = PPW // CHUNK
    HWsz = H * W
    G16 = CHUNK // L
    assert P % (NW * CHUNK) == 0 and NCHUNKS % 2 == 0
    assert HWsz % CHUNK == 0 and C % L == 0

    mesh = plsc.VectorSubcoreMesh(
        core_axis_name="c", subcore_axis_name="s", num_cores=NC, num_subcores=NS
    )

    @functools.partial(
        pl.kernel,
        out_type=jax.ShapeDtypeStruct((N, C, HWsz), jnp.float32),
        mesh=mesh,
        compiler_params=pltpu.CompilerParams(
            needs_layout_passes=False, use_tc_tiling_on_sc=False),
        scratch_types=[
            pltpu.VMEM((PPW,), jnp.float32),
            pltpu.VMEM((PPW,), jnp.float32),
            pltpu.VMEM((2, 4, CHUNK), jnp.int32),
            pltpu.VMEM((2, 4 * CHUNK), jnp.float32),
            pltpu.VMEM((2, 4, CHUNK, C), jnp.float32),
            pltpu.VMEM((2, C, CHUNK), jnp.float32),
            pltpu.SemaphoreType.DMA,
            pltpu.SemaphoreType.DMA,
            pltpu.SemaphoreType.DMA,
            pltpu.SemaphoreType.DMA,
        ],
    )
    def sc_call(gx_hbm, gy_hbm, table_hbm, out_hbm,
                gxw_v, gyw_v, idx_v, w_v, rows_v, out_v,
                rsem0, rsem1, osem0, osem1):
        cid = lax.axis_index("c")
        sid = lax.axis_index("s")
        wid = sid * NC + cid
        wbase = wid * PPW
        rsems = (rsem0, rsem1)
        osems = (osem0, osem1)

        pltpu.sync_copy(gx_hbm.at[pl.ds(wbase, PPW)], gxw_v)
        pltpu.sync_copy(gy_hbm.at[pl.ds(wbase, PPW)], gyw_v)

        def fire(g, b):
            nbase = ((wbase + g * CHUNK) // HWsz) * HWsz

            def coord_body(t, c2):
                gx = gxw_v[pl.ds(g * CHUNK + t * L, L)]
                gy = gyw_v[pl.ds(g * CHUNK + t * L, L)]
                ix = (gx + 1.0) * (W * 0.5) - 0.5
                iy = (gy + 1.0) * (H * 0.5) - 0.5
                ix0 = _floor_i32(ix)
                iy0 = _floor_i32(iy)
                wx1 = ix - ix0.astype(jnp.float32)
                wx0 = 1.0 - wx1
                wy1 = iy - iy0.astype(jnp.float32)
                wy0 = 1.0 - wy1
                for k, (dy, dx, wy, wx) in enumerate(
                    ((0, 0, wy0, wx0), (0, 1, wy0, wx1),
                     (1, 0, wy1, wx0), (1, 1, wy1, wx1))):
                    xi = ix0 + dx
                    yi = iy0 + dy
                    valid = ((xi >= 0) & (xi <= W - 1)
                             & (yi >= 0) & (yi <= H - 1))
                    xc = jnp.maximum(jnp.minimum(xi, W - 1), 0)
                    yc = jnp.maximum(jnp.minimum(yi, H - 1), 0)
                    idx_v[b, k, pl.ds(t * L, L)] = nbase + yc * W + xc
                    w_v[b, pl.ds(k * CHUNK + t * L, L)] = (
                        jnp.where(valid, wy * wx, 0.0))
                return c2

            lax.fori_loop(0, G16, coord_body, 0, unroll=False)
            for k in range(4):
                pltpu.async_copy(
                    table_hbm.at[idx_v.at[b, k]], rows_v.at[b, k], rsems[b])

        def drain_rows(b):
            for k in range(4):
                pltpu.make_async_copy(
                    table_hbm.at[idx_v.at[b, k]], rows_v.at[b, k],
                    rsems[b]).wait()

        row_iotas = [lax.iota(jnp.int32, L) + j * L for j in range(C // L)]

        def compute(g, b):
            def point_body(t, c2):
                tv = jnp.full((L,), t, jnp.int32)
                ws = [plsc.load_gather(w_v.at[b], [tv + (k * CHUNK)])
                      for k in range(4)]
                for j in range(C // L):
                    acc = ws[0] * rows_v[b, 0, t, pl.ds(j * L, L)]
                    acc += ws[1] * rows_v[b, 1, t, pl.ds(j * L, L)]
                    acc += ws[2] * rows_v[b, 2, t, pl.ds(j * L, L)]
                    acc += ws[3] * rows_v[b, 3, t, pl.ds(j * L, L)]
                    plsc.store_scatter(out_v.at[b], [row_iotas[j], tv], acc)
                return c2

            lax.fori_loop(0, CHUNK, point_body, 0, unroll=False)

        def out_dma(g, b):
            base = wbase + g * CHUNK
            n = base // HWsz
            pbase = base - n * HWsz
            return pltpu.async_copy(
                out_v.at[b], out_hbm.at[n, :, pl.ds(pbase, CHUNK)], osems[b])

        def drain_out(g, b):
            base = wbase + g * CHUNK
            n = base // HWsz
            pbase = base - n * HWsz
            pltpu.make_async_copy(
                out_v.at[b], out_hbm.at[n, :, pl.ds(pbase, CHUNK)],
                osems[b]).wait()

        fire(0, 0)
        fire(1, 1)

        def loop_body(i, carry):
            g0 = 2 * i

            def half(b):
                g = g0 + b
                drain_rows(b)

                @pl.when(i > 0)
                def _():
                    drain_out(g - 2, b)

                compute(g, b)
                out_dma(g, b)

                @pl.when(g + 2 < NCHUNKS)
                def _():
                    fire(g + 2, b)

            half(0)
            half(1)
            return carry

        lax.fori_loop(0, NCHUNKS // 2, loop_body, 0, unroll=False)
        drain_out(NCHUNKS - 2, 0)
        drain_out(NCHUNKS - 1, 1)

    return sc_call


@jax.jit
def kernel(x, grid):
    N, C, H, W = x.shape
    P = N * H * W
    table = jnp.transpose(x, (0, 2, 3, 1)).reshape(P, C)
    gx = grid[..., 0].reshape(P)
    gy = grid[..., 1].reshape(P)
    sc_call = _make_sc_call(N, C, H, W, NC=2, NS=16, CHUNK=96)
    out = sc_call(gx, gy, table)
    return out.reshape(N, C, H, W)


# speedup vs baseline: 1.3622x; 1.3622x over previous
import functools

import jax
import jax.numpy as jnp
from jax import lax
from jax.experimental import pallas as pl
from jax.experimental.pallas import tpu as pltpu
from jax.experimental.pallas import tpu_sc as plsc

L = 16


def _floor_i32(v):
    i = v.astype(jnp.int32)
    return jnp.where(i.astype(jnp.float32) > v, i - 1, i)


def _make_sc_call(N, C, H, W, NC, NS, CHUNK):
    P = N * H * W
    NW = NC * NS
    PPW = P // NW
    NCHUNKS = PPW // CHUNK
    HWsz = H * W
    G16 = CHUNK // L
    assert P % (NW * CHUNK) == 0 and NCHUNKS % 2 == 0
    assert HWsz % CHUNK == 0 and C % L == 0

    mesh = plsc.VectorSubcoreMesh(
        core_axis_name="c", subcore_axis_name="s", num_cores=NC, num_subcores=NS
    )

    @functools.partial(
        pl.kernel,
        out_type=jax.ShapeDtypeStruct((N, C, HWsz), jnp.float32),
        mesh=mesh,
        compiler_params=pltpu.CompilerParams(
            needs_layout_passes=False, use_tc_tiling_on_sc=False),
        scratch_types=[
            pltpu.VMEM((PPW,), jnp.float32),
            pltpu.VMEM((PPW,), jnp.float32),
            pltpu.VMEM((2, 4, CHUNK), jnp.int32),
            pltpu.VMEM((2, 4 * CHUNK), jnp.float32),
            pltpu.VMEM((2, 4, CHUNK, C), jnp.float32),
            pltpu.VMEM((2, C, CHUNK + 1), jnp.float32),
            pltpu.SemaphoreType.DMA,
            pltpu.SemaphoreType.DMA,
            pltpu.SemaphoreType.DMA,
            pltpu.SemaphoreType.DMA,
        ],
    )
    def sc_call(gx_hbm, gy_hbm, table_hbm, out_hbm,
                gxw_v, gyw_v, idx_v, w_v, rows_v, out_v,
                rsem0, rsem1, osem0, osem1):
        cid = lax.axis_index("c")
        sid = lax.axis_index("s")
        wid = sid * NC + cid
        wbase = wid * PPW
        rsems = (rsem0, rsem1)
        osems = (osem0, osem1)

        pltpu.sync_copy(gx_hbm.at[pl.ds(wbase, PPW)], gxw_v)
        pltpu.sync_copy(gy_hbm.at[pl.ds(wbase, PPW)], gyw_v)

        def fire(g, b):
            nbase = ((wbase + g * CHUNK) // HWsz) * HWsz

            def coord_body(t, c2):
                gx = gxw_v[pl.ds(g * CHUNK + t * L, L)]
                gy = gyw_v[pl.ds(g * CHUNK + t * L, L)]
                ix = (gx + 1.0) * (W * 0.5) - 0.5
                iy = (gy + 1.0) * (H * 0.5) - 0.5
                ix0 = _floor_i32(ix)
                iy0 = _floor_i32(iy)
                wx1 = ix - ix0.astype(jnp.float32)
                wx0 = 1.0 - wx1
                wy1 = iy - iy0.astype(jnp.float32)
                wy0 = 1.0 - wy1
                for k, (dy, dx, wy, wx) in enumerate(
                    ((0, 0, wy0, wx0), (0, 1, wy0, wx1),
                     (1, 0, wy1, wx0), (1, 1, wy1, wx1))):
                    xi = ix0 + dx
                    yi = iy0 + dy
                    valid = ((xi >= 0) & (xi <= W - 1)
                             & (yi >= 0) & (yi <= H - 1))
                    xc = jnp.maximum(jnp.minimum(xi, W - 1), 0)
                    yc = jnp.maximum(jnp.minimum(yi, H - 1), 0)
                    idx_v[b, k, pl.ds(t * L, L)] = nbase + yc * W + xc
                    w_v[b, pl.ds(k * CHUNK + t * L, L)] = (
                        jnp.where(valid, wy * wx, 0.0))
                return c2

            lax.fori_loop(0, G16, coord_body, 0, unroll=False)
            for k in range(4):
                pltpu.async_copy(
                    table_hbm.at[idx_v.at[b, k]], rows_v.at[b, k], rsems[b])

        def drain_rows(b):
            for k in range(4):
                pltpu.make_async_copy(
                    table_hbm.at[idx_v.at[b, k]], rows_v.at[b, k],
                    rsems[b]).wait()

        row_iotas = [lax.iota(jnp.int32, L) + j * L for j in range(C // L)]

        def compute(g, b):
            def point_body(t, c2):
                tv = jnp.full((L,), t, jnp.int32)
                ws = [plsc.load_gather(w_v.at[b], [tv + (k * CHUNK)])
                      for k in range(4)]
                for j in range(C // L):
                    acc = ws[0] * rows_v[b, 0, t, pl.ds(j * L, L)]
                    acc += ws[1] * rows_v[b, 1, t, pl.ds(j * L, L)]
                    acc += ws[2] * rows_v[b, 2, t, pl.ds(j * L, L)]
                    acc += ws[3] * rows_v[b, 3, t, pl.ds(j * L, L)]
                    plsc.store_scatter(out_v.at[b], [row_iotas[j], tv], acc)
                return c2

            lax.fori_loop(0, CHUNK, point_body, 0, unroll=False)

        def out_dma(g, b):
            base = wbase + g * CHUNK
            n = base // HWsz
            pbase = base - n * HWsz
            return pltpu.async_copy(
                out_v.at[b, :, pl.ds(0, CHUNK)],
                out_hbm.at[n, :, pl.ds(pbase, CHUNK)], osems[b])

        def drain_out(g, b):
            base = wbase + g * CHUNK
            n = base // HWsz
            pbase = base - n * HWsz
            pltpu.make_async_copy(
                out_v.at[b, :, pl.ds(0, CHUNK)],
                out_hbm.at[n, :, pl.ds(pbase, CHUNK)],
                osems[b]).wait()

        fire(0, 0)
        fire(1, 1)

        def loop_body(i, carry):
            g0 = 2 * i

            def half(b):
                g = g0 + b
                drain_rows(b)

                @pl.when(i > 0)
                def _():
                    drain_out(g - 2, b)

                compute(g, b)
                out_dma(g, b)

                @pl.when(g + 2 < NCHUNKS)
                def _():
                    fire(g + 2, b)

            half(0)
            half(1)
            return carry

        lax.fori_loop(0, NCHUNKS // 2, loop_body, 0, unroll=False)
        drain_out(NCHUNKS - 2, 0)
        drain_out(NCHUNKS - 1, 1)

    return sc_call


@jax.jit
def kernel(x, grid):
    N, C, H, W = x.shape
    P = N * H * W
    table = jnp.transpose(x, (0, 2, 3, 1)).reshape(P, C)
    gx = grid[..., 0].reshape(P)
    gy = grid[..., 1].reshape(P)
    sc_call = _make_sc_call(N, C, H, W, NC=2, NS=16, CHUNK=96)
    out = sc_call(gx, gy, table)
    return out.reshape(N, C, H, W)


# speedup vs baseline: 2.1209x; 1.5569x over previous
import functools

import jax
import jax.numpy as jnp
from jax import lax
from jax.experimental import pallas as pl
from jax.experimental.pallas import tpu as pltpu
from jax.experimental.pallas import tpu_sc as plsc

L = 16


def _floor_i32(v):
    i = v.astype(jnp.int32)
    return jnp.where(i.astype(jnp.float32) > v, i - 1, i)


def _make_sc_call(N, C, H, W, NC, NS, CHUNK):
    P = N * H * W
    NW = NC * NS
    PPW = P // NW
    NCHUNKS = PPW // CHUNK
    HWsz = H * W
    G16 = CHUNK // L
    assert P % (NW * CHUNK) == 0 and NCHUNKS % 2 == 0
    assert HWsz % CHUNK == 0 and C % L == 0

    mesh = plsc.VectorSubcoreMesh(
        core_axis_name="c", subcore_axis_name="s", num_cores=NC, num_subcores=NS
    )

    @functools.partial(
        pl.kernel,
        out_type=jax.ShapeDtypeStruct((N, C, HWsz), jnp.float32),
        mesh=mesh,
        compiler_params=pltpu.CompilerParams(
            needs_layout_passes=False, use_tc_tiling_on_sc=False),
        scratch_types=[
            pltpu.VMEM((PPW,), jnp.float32),
            pltpu.VMEM((PPW,), jnp.float32),
            pltpu.VMEM((2, 4, CHUNK), jnp.int32),
            pltpu.VMEM((2, 4 * CHUNK), jnp.float32),
            pltpu.VMEM((2, 4, CHUNK, C), jnp.float32),
            pltpu.VMEM((2, C, CHUNK + 1), jnp.float32),
            pltpu.SemaphoreType.DMA,
            pltpu.SemaphoreType.DMA,
            pltpu.SemaphoreType.DMA,
            pltpu.SemaphoreType.DMA,
        ],
    )
    def sc_call(gx_hbm, gy_hbm, table_hbm, out_hbm,
                gxw_v, gyw_v, idx_v, w_v, rows_v, out_v,
                rsem0, rsem1, osem0, osem1):
        cid = lax.axis_index("c")
        sid = lax.axis_index("s")
        wid = sid * NC + cid
        wbase = wid * PPW
        rsems = (rsem0, rsem1)
        osems = (osem0, osem1)

        pltpu.sync_copy(gx_hbm.at[pl.ds(wbase, PPW)], gxw_v)
        pltpu.sync_copy(gy_hbm.at[pl.ds(wbase, PPW)], gyw_v)

        def fire(g, b):
            nbase = ((wbase + g * CHUNK) // HWsz) * HWsz

            def coord_body(t, c2):
                gx = gxw_v[pl.ds(g * CHUNK + t * L, L)]
                gy = gyw_v[pl.ds(g * CHUNK + t * L, L)]
                ix = (gx + 1.0) * (W * 0.5) - 0.5
                iy = (gy + 1.0) * (H * 0.5) - 0.5
                ix0 = _floor_i32(ix)
                iy0 = _floor_i32(iy)
                wx1 = ix - ix0.astype(jnp.float32)
                wx0 = 1.0 - wx1
                wy1 = iy - iy0.astype(jnp.float32)
                wy0 = 1.0 - wy1
                for k, (dy, dx, wy, wx) in enumerate(
                    ((0, 0, wy0, wx0), (0, 1, wy0, wx1),
                     (1, 0, wy1, wx0), (1, 1, wy1, wx1))):
                    xi = ix0 + dx
                    yi = iy0 + dy
                    valid = ((xi >= 0) & (xi <= W - 1)
                             & (yi >= 0) & (yi <= H - 1))
                    xc = jnp.maximum(jnp.minimum(xi, W - 1), 0)
                    yc = jnp.maximum(jnp.minimum(yi, H - 1), 0)
                    idx_v[b, k, pl.ds(t * L, L)] = nbase + yc * W + xc
                    w_v[b, pl.ds(k * CHUNK + t * L, L)] = (
                        jnp.where(valid, wy * wx, 0.0))
                return c2

            lax.fori_loop(0, G16, coord_body, 0, unroll=False)
            for k in range(4):
                pltpu.async_copy(
                    table_hbm.at[idx_v.at[b, k]], rows_v.at[b, k], rsems[b])

        def drain_rows(b):
            for k in range(4):
                pltpu.make_async_copy(
                    table_hbm.at[idx_v.at[b, k]], rows_v.at[b, k],
                    rsems[b]).wait()

        row_iotas = [lax.iota(jnp.int32, L) + j * L for j in range(C // L)]

        def compute(g, b):
            @functools.partial(plsc.parallel_loop, 0, CHUNK, unroll=2)
            def point_body(t):
                tv = jnp.full((L,), t, jnp.int32)
                ws = [plsc.load_gather(w_v.at[b], [tv + (k * CHUNK)])
                      for k in range(4)]
                for j in range(C // L):
                    acc = ws[0] * rows_v[b, 0, t, pl.ds(j * L, L)]
                    acc += ws[1] * rows_v[b, 1, t, pl.ds(j * L, L)]
                    acc += ws[2] * rows_v[b, 2, t, pl.ds(j * L, L)]
                    acc += ws[3] * rows_v[b, 3, t, pl.ds(j * L, L)]
                    plsc.store_scatter(out_v.at[b], [row_iotas[j], tv], acc)

        def out_dma(g, b):
            base = wbase + g * CHUNK
            n = base // HWsz
            pbase = base - n * HWsz
            return pltpu.async_copy(
                out_v.at[b, :, pl.ds(0, CHUNK)],
                out_hbm.at[n, :, pl.ds(pbase, CHUNK)], osems[b])

        def drain_out(g, b):
            base = wbase + g * CHUNK
            n = base // HWsz
            pbase = base - n * HWsz
            pltpu.make_async_copy(
                out_v.at[b, :, pl.ds(0, CHUNK)],
                out_hbm.at[n, :, pl.ds(pbase, CHUNK)],
                osems[b]).wait()

        fire(0, 0)
        fire(1, 1)

        def loop_body(i, carry):
            g0 = 2 * i

            def half(b):
                g = g0 + b
                drain_rows(b)

                @pl.when(i > 0)
                def _():
                    drain_out(g - 2, b)

                compute(g, b)
                out_dma(g, b)

                @pl.when(g + 2 < NCHUNKS)
                def _():
                    fire(g + 2, b)

            half(0)
            half(1)
            return carry

        lax.fori_loop(0, NCHUNKS // 2, loop_body, 0, unroll=False)
        drain_out(NCHUNKS - 2, 0)
        drain_out(NCHUNKS - 1, 1)

    return sc_call


@jax.jit
def kernel(x, grid):
    N, C, H, W = x.shape
    P = N * H * W
    table = jnp.transpose(x, (0, 2, 3, 1)).reshape(P, C)
    gx = grid[..., 0].reshape(P)
    gy = grid[..., 1].reshape(P)
    sc_call = _make_sc_call(N, C, H, W, NC=2, NS=16, CHUNK=96)
    out = sc_call(gx, gy, table)
    return out.reshape(N, C, H, W)
